# Initial kernel scaffold; baseline (speedup 1.0000x reference)
import jax
import jax.numpy as jnp
from jax.experimental import pallas as pl


def kernel(x, weight, loop_weight, h_bias, edge_index_fwd, edge_index_bwd):
    raise NotImplementedError("write your pallas kernel here")


# speedup vs baseline: 2.0034x; 2.0034x over previous
import functools

import jax
import jax.numpy as jnp
from jax import lax
from jax.experimental import pallas as pl
from jax.experimental.pallas import tpu as pltpu
from jax.experimental.pallas import tpu_sc as plsc

N_NODES = 100000
N_EDGES = 1600000
D = 256

NUM_SLABS = 16
SLAB = 16
NT = 16
CHUNK_ROWS = 8
SUPER = 98
TILE_EDGE_ROWS = SUPER * CHUNK_ROWS
EDGE_ROWS = NT * TILE_EDGE_ROWS
E_PAD = EDGE_ROWS * 128
NROW_PAD = 100352
TROWS = NROW_PAD // NT
ZCH = 224


def _sc_aggregate(xr, src_all, dst_all):
    mesh = plsc.VectorSubcoreMesh(core_axis_name="c", subcore_axis_name="s")

    @functools.partial(
        pl.kernel,
        mesh=mesh,
        compiler_params=pltpu.CompilerParams(use_tc_tiling_on_sc=False),
        out_type=[
            jax.ShapeDtypeStruct((2, NUM_SLABS, NROW_PAD, SLAB), jnp.float32),
            jax.ShapeDtypeStruct((2, NROW_PAD, SLAB), jnp.float32),
        ],
        scratch_types=[
            pltpu.VMEM((CHUNK_ROWS, 128), jnp.int32),
            pltpu.VMEM((CHUNK_ROWS, 128), jnp.int32),
            pltpu.VMEM((CHUNK_ROWS, 128), jnp.int32),
            pltpu.VMEM((CHUNK_ROWS, 128, SLAB), jnp.float32),
            pltpu.VMEM((ZCH, SLAB), jnp.float32),
            pltpu.VMEM((128, SLAB), jnp.float32),
            pltpu.VMEM((ZCH, SLAB), jnp.float32),
            pltpu.VMEM_SHARED((NROW_PAD, SLAB), jnp.float32),
            pltpu.SemaphoreType.DMA,
        ],
    )
    def sc_kernel(xr_hbm, src_hbm, dst_hbm, agg_out, deg_out,
                  src_v, sidx_v, dst_v, rows_v, zero_rows, ones_rows,
                  stage_v, shared_agg, sem):
        c = lax.axis_index("c")
        tid = lax.axis_index("s")
        row_base = tid * TROWS

        def fill_zr(i, _):
            zero_rows[i, :] = jnp.zeros((SLAB,), jnp.float32)
            return 0
        lax.fori_loop(0, ZCH, fill_zr, 0)

        def fill_on(i, _):
            ones_rows[i, :] = jnp.ones((SLAB,), jnp.float32)
            return 0
        lax.fori_loop(0, 128, fill_on, 0)

        def zero_phase():
            for z in range(TROWS // ZCH):
                pltpu.sync_copy(zero_rows,
                                shared_agg.at[pl.ds(row_base + z * ZCH, ZCH)])

        def write_phase(out_view):
            for k in range(TROWS // ZCH):
                off = row_base + k * ZCH
                pltpu.sync_copy(shared_agg.at[pl.ds(off, ZCH)], stage_v)
                pltpu.sync_copy(stage_v, out_view.at[pl.ds(off, ZCH)])

        def pass_body(p, _):
            r = p // 8
            s = c * 8 + (p % 8)

            zero_phase()
            plsc.subcore_barrier()

            def chunk_body(i, _):
                row0 = (tid * SUPER + i) * CHUNK_ROWS
                pltpu.sync_copy(src_hbm.at[r, pl.ds(row0, CHUNK_ROWS)], src_v)
                pltpu.sync_copy(dst_hbm.at[r, pl.ds(row0, CHUNK_ROWS)], dst_v)
                for j in range(CHUNK_ROWS):
                    for l in range(8):
                        sl = pl.ds(l * 16, 16)
                        sidx_v[j, sl] = src_v[j, sl] + s
                cps = [
                    pltpu.async_copy(xr_hbm.at[sidx_v.at[j]], rows_v.at[j], sem)
                    for j in range(CHUNK_ROWS)
                ]
                for cp in cps:
                    cp.wait()
                for j in range(CHUNK_ROWS):
                    pltpu.sync_copy(rows_v.at[j],
                                    shared_agg.at[dst_v.at[j]], add=True)
                return 0
            lax.fori_loop(0, SUPER, chunk_body, 0)

            plsc.subcore_barrier()
            write_phase(agg_out.at[r, s])
            plsc.subcore_barrier()
            return 0

        lax.fori_loop(0, 16, pass_body, 0)

        zero_phase()
        plsc.subcore_barrier()

        def deg_body(i, _):
            row0 = (tid * SUPER + i) * CHUNK_ROWS
            pltpu.sync_copy(dst_hbm.at[c, pl.ds(row0, CHUNK_ROWS)], dst_v)
            for j in range(CHUNK_ROWS):
                pltpu.sync_copy(ones_rows,
                                shared_agg.at[dst_v.at[j]], add=True)
            return 0
        lax.fori_loop(0, SUPER, deg_body, 0)

        plsc.subcore_barrier()
        write_phase(deg_out.at[c])

    return sc_kernel(xr, src_all, dst_all)


def _tc_body(aggf, aggb, xb, degf, degb, w0, w1, wl, bias, out):
    nf = 1.0 / jnp.maximum(degf[...], 1.0)
    nb = 1.0 / jnp.maximum(degb[...], 1.0)
    acc = jnp.dot(aggf[...] * nf, w0[...], preferred_element_type=jnp.float32)
    acc = acc + jnp.dot(aggb[...] * nb, w1[...], preferred_element_type=jnp.float32)
    acc = acc + jnp.dot(xb[...], wl[...], preferred_element_type=jnp.float32)
    out[...] = jnp.maximum(acc + bias[...], 0.0)


def _tc_fused(aggf, aggb, x, degf, degb, w0, w1, wl, bias):
    rb = 1000
    nblk = N_NODES // rb
    row = pl.BlockSpec((rb, D), lambda i: (i, 0))
    dcol = pl.BlockSpec((rb, 1), lambda i: (i, 0))
    full = pl.BlockSpec((D, D), lambda i: (0, 0))
    brow = pl.BlockSpec((1, D), lambda i: (0, 0))
    return pl.pallas_call(
        _tc_body,
        grid=(nblk,),
        in_specs=[row, row, row, dcol, dcol, full, full, full, brow],
        out_specs=row,
        out_shape=jax.ShapeDtypeStruct((N_NODES, D), jnp.float32),
    )(aggf, aggb, x, degf, degb, w0, w1, wl, bias)


def kernel(x, weight, loop_weight, h_bias, edge_index_fwd, edge_index_bwd):
    n = x.shape[0]
    pad = E_PAD - N_EDGES

    def prep(ei):
        src16 = ei[0].astype(jnp.int32) * NUM_SLABS
        dst = ei[1].astype(jnp.int32)
        src16 = jnp.concatenate([src16, jnp.zeros((pad,), jnp.int32)])
        dst = jnp.concatenate(
            [dst, n + (jnp.arange(pad, dtype=jnp.int32) % 128)])
        return src16.reshape(EDGE_ROWS, 128), dst.reshape(EDGE_ROWS, 128)

    sf, df = prep(edge_index_fwd)
    sb, db = prep(edge_index_bwd)
    src_all = jnp.stack([sf, sb])
    dst_all = jnp.stack([df, db])
    xr = x.reshape(n * NUM_SLABS, SLAB)

    agg_sl, deg = _sc_aggregate(xr, src_all, dst_all)
    agg = agg_sl[:, :, :n, :].transpose(0, 2, 1, 3).reshape(2, n, D)
    degf = deg[0, :n, 0].reshape(n, 1)
    degb = deg[1, :n, 0].reshape(n, 1)
    return _tc_fused(agg[0], agg[1], x, degf, degb,
                     weight[0], weight[1], loop_weight,
                     h_bias.reshape(1, D))


# speedup vs baseline: 2.4718x; 1.2338x over previous
import functools

import jax
import jax.numpy as jnp
from jax import lax
from jax.experimental import pallas as pl
from jax.experimental.pallas import tpu as pltpu
from jax.experimental.pallas import tpu_sc as plsc

N_NODES = 100000
N_EDGES = 1600000
D = 256

NUM_SLABS = 16
SLAB = 16
NT = 16
CHUNK_ROWS = 4
SUPER = 196
TILE_EDGE_ROWS = SUPER * CHUNK_ROWS
EDGE_ROWS = NT * TILE_EDGE_ROWS
E_PAD = EDGE_ROWS * 128
NROW_PAD = 100352
TROWS = NROW_PAD // NT
ZCH = 224
WCH = 112


def _sc_aggregate(xr, src_all, dst_all):
    mesh = plsc.VectorSubcoreMesh(core_axis_name="c", subcore_axis_name="s")

    @functools.partial(
        pl.kernel,
        mesh=mesh,
        compiler_params=pltpu.CompilerParams(use_tc_tiling_on_sc=False),
        out_type=[
            jax.ShapeDtypeStruct((2, NUM_SLABS, NROW_PAD, SLAB), jnp.float32),
            jax.ShapeDtypeStruct((2, NROW_PAD, SLAB), jnp.float32),
        ],
        scratch_types=[
            pltpu.VMEM((2, CHUNK_ROWS, 128), jnp.int32),
            pltpu.VMEM((CHUNK_ROWS, 128), jnp.int32),
            pltpu.VMEM((4, CHUNK_ROWS, 128), jnp.int32),
            pltpu.VMEM((2, CHUNK_ROWS, 128, SLAB), jnp.float32),
            pltpu.VMEM((ZCH, SLAB), jnp.float32),
            pltpu.VMEM((128, SLAB), jnp.float32),
            pltpu.VMEM((2, WCH, SLAB), jnp.float32),
            pltpu.VMEM_SHARED((NROW_PAD, SLAB), jnp.float32),
            pltpu.SemaphoreType.DMA,
            pltpu.SemaphoreType.DMA,
            pltpu.SemaphoreType.DMA((2,)),
            pltpu.SemaphoreType.DMA,
            pltpu.SemaphoreType.DMA,
        ],
    )
    def sc_kernel(xr_hbm, src_hbm, dst_hbm, agg_out, deg_out,
                  src_v, sidx_v, dst_v, rows_v, zero_rows, ones_rows,
                  stage_v, shared_agg, isem, gsem, scsem, rsem, wsem):
        c = lax.axis_index("c")
        tid = lax.axis_index("s")
        row_base = tid * TROWS

        def fill_zr(i, _):
            zero_rows[i, :] = jnp.zeros((SLAB,), jnp.float32)
            return 0
        lax.fori_loop(0, ZCH, fill_zr, 0)

        def fill_on(i, _):
            ones_rows[i, :] = jnp.ones((SLAB,), jnp.float32)
            return 0
        lax.fori_loop(0, 128, fill_on, 0)

        def zero_phase():
            for z in range(TROWS // ZCH):
                pltpu.sync_copy(zero_rows,
                                shared_agg.at[pl.ds(row_base + z * ZCH, ZCH)])

        def write_phase(out_view):
            nchunk = TROWS // WCH

            def rd(k, kb):
                pltpu.async_copy(
                    shared_agg.at[pl.ds(row_base + k * WCH, WCH)],
                    stage_v.at[kb], rsem)

            def wr_desc(k, kb):
                return pltpu.make_async_copy(
                    stage_v.at[kb], out_view.at[pl.ds(row_base + k * WCH, WCH)],
                    wsem)

            rd(0, 0)

            def wbody(k, _):
                kb = k % 2
                pltpu.make_async_copy(
                    shared_agg.at[pl.ds(row_base, WCH)], stage_v.at[kb],
                    rsem).wait()

                @pl.when(k >= 2)
                def _():
                    wr_desc(k, kb).wait()
                rd((k + 1) % nchunk, 1 - kb)
                wr_desc(k, kb).start()
                return 0
            lax.fori_loop(0, nchunk, wbody, 0)
            wr_desc(nchunk - 2, 0).wait()
            wr_desc(nchunk - 1, 1).wait()
            pltpu.make_async_copy(
                shared_agg.at[pl.ds(row_base, WCH)], stage_v.at[0], rsem).wait()

        def idx_load(rel, i, use_src):
            sc = i % SUPER
            row0 = (tid * SUPER + sc) * CHUNK_ROWS

            @pl.when(use_src)
            def _():
                pltpu.async_copy(src_hbm.at[rel, pl.ds(row0, CHUNK_ROWS)],
                                 src_v.at[i % 2], isem)
            pltpu.async_copy(dst_hbm.at[rel, pl.ds(row0, CHUNK_ROWS)],
                             dst_v.at[i % 4], isem)

        def idx_wait(use_src):
            @pl.when(use_src)
            def _():
                pltpu.make_async_copy(
                    src_hbm.at[0, pl.ds(0, CHUNK_ROWS)], src_v.at[0],
                    isem).wait()
            pltpu.make_async_copy(
                dst_hbm.at[0, pl.ds(0, CHUNK_ROWS)], dst_v.at[0], isem).wait()

        def sc_wait(b):
            for j in range(CHUNK_ROWS):
                pltpu.make_async_copy(
                    rows_v.at[0, j], shared_agg.at[dst_v.at[0, j]],
                    scsem.at[b]).wait()

        def edge_loop(rel, s, gather_on):
            idx_load(rel, 0, gather_on)

            def body(i, _):
                b = i % 2
                d4 = i % 4
                idx_wait(gather_on)
                idx_load(rel, i + 1, gather_on)

                @pl.when(i >= 2)
                def _():
                    sc_wait(b)
                if gather_on:
                    for j in range(CHUNK_ROWS):
                        for l in range(8):
                            sl = pl.ds(l * 16, 16)
                            sidx_v[j, sl] = src_v[b, j, sl] + s
                    for j in range(CHUNK_ROWS):
                        pltpu.async_copy(xr_hbm.at[sidx_v.at[j]],
                                         rows_v.at[b, j], gsem)
                    for j in range(CHUNK_ROWS):
                        pltpu.make_async_copy(
                            xr_hbm.at[sidx_v.at[j]], rows_v.at[b, j],
                            gsem).wait()
                    for j in range(CHUNK_ROWS):
                        pltpu.async_copy(rows_v.at[b, j],
                                         shared_agg.at[dst_v.at[d4, j]],
                                         scsem.at[b], add=True)
                else:
                    for j in range(CHUNK_ROWS):
                        pltpu.async_copy(ones_rows,
                                         shared_agg.at[dst_v.at[d4, j]],
                                         scsem.at[b], add=True)
                return 0
            lax.fori_loop(0, SUPER, body, 0)
            sc_wait(0)
            sc_wait(1)
            idx_wait(gather_on)

        def pass_body(p, _):
            r = p // 8
            s = c * 8 + (p % 8)
            zero_phase()
            plsc.subcore_barrier()
            edge_loop(r, s, True)
            plsc.subcore_barrier()
            write_phase(agg_out.at[r, s])
            plsc.subcore_barrier()
            return 0

        lax.fori_loop(0, 16, pass_body, 0)

        zero_phase()
        plsc.subcore_barrier()
        edge_loop(c, 0, False)
        plsc.subcore_barrier()
        write_phase(deg_out.at[c])

    return sc_kernel(xr, src_all, dst_all)


def _tc_body(aggf, aggb, xb, degf, degb, w0, w1, wl, bias, out):
    nf = 1.0 / jnp.maximum(degf[...], 1.0)
    nb = 1.0 / jnp.maximum(degb[...], 1.0)
    acc = jnp.dot(aggf[...] * nf, w0[...], preferred_element_type=jnp.float32)
    acc = acc + jnp.dot(aggb[...] * nb, w1[...], preferred_element_type=jnp.float32)
    acc = acc + jnp.dot(xb[...], wl[...], preferred_element_type=jnp.float32)
    out[...] = jnp.maximum(acc + bias[...], 0.0)


def _tc_fused(aggf, aggb, x, degf, degb, w0, w1, wl, bias):
    rb = 1000
    nblk = N_NODES // rb
    row = pl.BlockSpec((rb, D), lambda i: (i, 0))
    dcol = pl.BlockSpec((rb, 1), lambda i: (i, 0))
    full = pl.BlockSpec((D, D), lambda i: (0, 0))
    brow = pl.BlockSpec((1, D), lambda i: (0, 0))
    return pl.pallas_call(
        _tc_body,
        grid=(nblk,),
        in_specs=[row, row, row, dcol, dcol, full, full, full, brow],
        out_specs=row,
        out_shape=jax.ShapeDtypeStruct((N_NODES, D), jnp.float32),
    )(aggf, aggb, x, degf, degb, w0, w1, wl, bias)


def kernel(x, weight, loop_weight, h_bias, edge_index_fwd, edge_index_bwd):
    n = x.shape[0]
    pad = E_PAD - N_EDGES

    def prep(ei):
        src16 = ei[0].astype(jnp.int32) * NUM_SLABS
        dst = ei[1].astype(jnp.int32)
        src16 = jnp.concatenate([src16, jnp.zeros((pad,), jnp.int32)])
        dst = jnp.concatenate(
            [dst, n + (jnp.arange(pad, dtype=jnp.int32) % 128)])
        return src16.reshape(EDGE_ROWS, 128), dst.reshape(EDGE_ROWS, 128)

    sf, df = prep(edge_index_fwd)
    sb, db = prep(edge_index_bwd)
    src_all = jnp.stack([sf, sb])
    dst_all = jnp.stack([df, db])
    xr = x.reshape(n * NUM_SLABS, SLAB)

    agg_sl, deg = _sc_aggregate(xr, src_all, dst_all)
    agg = agg_sl[:, :, :n, :].transpose(0, 2, 1, 3).reshape(2, n, D)
    degf = deg[0, :n, 0].reshape(n, 1)
    degb = deg[1, :n, 0].reshape(n, 1)
    return _tc_fused(agg[0], agg[1], x, degf, degb,
                     weight[0], weight[1], loop_weight,
                     h_bias.reshape(1, D))


# speedup vs baseline: 3.0287x; 1.2253x over previous
import functools

import jax
import jax.numpy as jnp
from jax import lax
from jax.experimental import pallas as pl
from jax.experimental.pallas import tpu as pltpu
from jax.experimental.pallas import tpu_sc as plsc

N_NODES = 100000
N_EDGES = 1600000
D = 256

NUM_SLABS = 16
SLAB = 16
NT = 16
CHUNK_ROWS = 4
SUPER = 196
TILE_EDGE_ROWS = SUPER * CHUNK_ROWS
EDGE_ROWS = NT * TILE_EDGE_ROWS
E_PAD = EDGE_ROWS * 128
NROW_PAD = 100352
TROWS = NROW_PAD // NT
ZCH = 224
WCH = 112


def _sc_aggregate(xr, src_all, dst_all):
    mesh = plsc.VectorSubcoreMesh(core_axis_name="c", subcore_axis_name="s")

    @functools.partial(
        pl.kernel,
        mesh=mesh,
        compiler_params=pltpu.CompilerParams(use_tc_tiling_on_sc=False),
        out_type=[
            jax.ShapeDtypeStruct((2, NUM_SLABS, NROW_PAD, SLAB), jnp.float32),
            jax.ShapeDtypeStruct((2, NROW_PAD, SLAB), jnp.float32),
        ],
        scratch_types=[
            pltpu.VMEM((2, CHUNK_ROWS, 128), jnp.int32),
            pltpu.VMEM((2, CHUNK_ROWS, 128), jnp.int32),
            pltpu.VMEM((4, CHUNK_ROWS, 128), jnp.int32),
            pltpu.VMEM((2, CHUNK_ROWS, 128, SLAB), jnp.float32),
            pltpu.VMEM((ZCH, SLAB), jnp.float32),
            pltpu.VMEM((128, SLAB), jnp.float32),
            pltpu.VMEM((2, WCH, SLAB), jnp.float32),
            pltpu.VMEM_SHARED((NROW_PAD, SLAB), jnp.float32),
            pltpu.SemaphoreType.DMA,
            pltpu.SemaphoreType.DMA((2,)),
            pltpu.SemaphoreType.DMA((2,)),
            pltpu.SemaphoreType.DMA,
            pltpu.SemaphoreType.DMA,
        ],
    )
    def sc_kernel(xr_hbm, src_hbm, dst_hbm, agg_out, deg_out,
                  src_v, sidx_v, dst_v, rows_v, zero_rows, ones_rows,
                  stage_v, shared_agg, isem, gsem, scsem, rsem, wsem):
        c = lax.axis_index("c")
        tid = lax.axis_index("s")
        row_base = tid * TROWS

        def fill_zr(i, _):
            zero_rows[i, :] = jnp.zeros((SLAB,), jnp.float32)
            return 0
        lax.fori_loop(0, ZCH, fill_zr, 0)

        def fill_on(i, _):
            ones_rows[i, :] = jnp.ones((SLAB,), jnp.float32)
            return 0
        lax.fori_loop(0, 128, fill_on, 0)

        def zero_phase():
            for z in range(TROWS // ZCH):
                pltpu.async_copy(zero_rows,
                                 shared_agg.at[pl.ds(row_base + z * ZCH, ZCH)],
                                 rsem)
            for z in range(TROWS // ZCH):
                pltpu.make_async_copy(
                    zero_rows, shared_agg.at[pl.ds(row_base, ZCH)], rsem).wait()

        def write_phase(out_view):
            nchunk = TROWS // WCH

            def rd(k, kb):
                pltpu.async_copy(
                    shared_agg.at[pl.ds(row_base + k * WCH, WCH)],
                    stage_v.at[kb], rsem)

            def wr_desc(k, kb):
                return pltpu.make_async_copy(
                    stage_v.at[kb], out_view.at[pl.ds(row_base + k * WCH, WCH)],
                    wsem)

            rd(0, 0)

            def wbody(k, _):
                kb = k % 2
                pltpu.make_async_copy(
                    shared_agg.at[pl.ds(row_base, WCH)], stage_v.at[kb],
                    rsem).wait()

                @pl.when(k >= 2)
                def _():
                    wr_desc(k, kb).wait()
                rd((k + 1) % nchunk, 1 - kb)
                wr_desc(k, kb).start()
                return 0
            lax.fori_loop(0, nchunk, wbody, 0)
            wr_desc(nchunk - 2, 0).wait()
            wr_desc(nchunk - 1, 1).wait()
            pltpu.make_async_copy(
                shared_agg.at[pl.ds(row_base, WCH)], stage_v.at[0], rsem).wait()

        def idx_load(rel, i, use_src):
            sc = i % SUPER
            row0 = (tid * SUPER + sc) * CHUNK_ROWS

            @pl.when(use_src)
            def _():
                pltpu.async_copy(src_hbm.at[rel, pl.ds(row0, CHUNK_ROWS)],
                                 src_v.at[i % 2], isem)
            pltpu.async_copy(dst_hbm.at[rel, pl.ds(row0, CHUNK_ROWS)],
                             dst_v.at[i % 4], isem)

        def idx_wait(use_src):
            @pl.when(use_src)
            def _():
                pltpu.make_async_copy(
                    src_hbm.at[0, pl.ds(0, CHUNK_ROWS)], src_v.at[0],
                    isem).wait()
            pltpu.make_async_copy(
                dst_hbm.at[0, pl.ds(0, CHUNK_ROWS)], dst_v.at[0], isem).wait()

        def sc_wait(b):
            for j in range(CHUNK_ROWS):
                pltpu.make_async_copy(
                    rows_v.at[0, j], shared_agg.at[dst_v.at[0, j]],
                    scsem.at[b]).wait()

        def edge_loop(rel, s, gather_on):
            idx_load(rel, 0, gather_on)

            def g_wait(b):
                for j in range(CHUNK_ROWS):
                    pltpu.make_async_copy(
                        xr_hbm.at[sidx_v.at[0, j]], rows_v.at[0, j],
                        gsem.at[b]).wait()

            def scatter_issue(b, d4):
                for j in range(CHUNK_ROWS):
                    pltpu.async_copy(rows_v.at[b, j],
                                     shared_agg.at[dst_v.at[d4, j]],
                                     scsem.at[b], add=True)

            def body(i, _):
                b = i % 2
                d4 = i % 4
                idx_wait(gather_on)
                idx_load(rel, i + 1, gather_on)

                @pl.when(i >= 2)
                def _():
                    sc_wait(b)
                if gather_on:
                    for j in range(CHUNK_ROWS):
                        for l in range(8):
                            sl = pl.ds(l * 16, 16)
                            sidx_v[b, j, sl] = src_v[b, j, sl] + s
                    for j in range(CHUNK_ROWS):
                        pltpu.async_copy(xr_hbm.at[sidx_v.at[b, j]],
                                         rows_v.at[b, j], gsem.at[b])

                    @pl.when(i >= 1)
                    def _():
                        g_wait(1 - b)
                        scatter_issue(1 - b, (i + 3) % 4)
                else:
                    for j in range(CHUNK_ROWS):
                        pltpu.async_copy(ones_rows,
                                         shared_agg.at[dst_v.at[d4, j]],
                                         scsem.at[b], add=True)
                return 0
            lax.fori_loop(0, SUPER, body, 0)
            if gather_on:
                lb = (SUPER - 1) % 2
                g_wait(lb)
                scatter_issue(lb, (SUPER - 1) % 4)
            sc_wait(0)
            sc_wait(1)
            idx_wait(gather_on)

        def pass_body(p, _):
            r = p // 8
            s = c * 8 + (p % 8)
            zero_phase()
            plsc.subcore_barrier()
            edge_loop(r, s, True)
            plsc.subcore_barrier()
            write_phase(agg_out.at[r, s])
            plsc.subcore_barrier()
            return 0

        lax.fori_loop(0, 16, pass_body, 0)

        zero_phase()
        plsc.subcore_barrier()
        edge_loop(c, 0, False)
        plsc.subcore_barrier()
        write_phase(deg_out.at[c])

    return sc_kernel(xr, src_all, dst_all)


def _tc_body(aggf, aggb, xb, degf, degb, w0, w1, wl, bias, out):
    nf = 1.0 / jnp.maximum(degf[...], 1.0)
    nb = 1.0 / jnp.maximum(degb[...], 1.0)
    acc = jnp.dot(aggf[...] * nf, w0[...], preferred_element_type=jnp.float32)
    acc = acc + jnp.dot(aggb[...] * nb, w1[...], preferred_element_type=jnp.float32)
    acc = acc + jnp.dot(xb[...], wl[...], preferred_element_type=jnp.float32)
    out[...] = jnp.maximum(acc + bias[...], 0.0)


def _tc_fused(aggf, aggb, x, degf, degb, w0, w1, wl, bias):
    rb = 1000
    nblk = N_NODES // rb
    row = pl.BlockSpec((rb, D), lambda i: (i, 0))
    dcol = pl.BlockSpec((rb, 1), lambda i: (i, 0))
    full = pl.BlockSpec((D, D), lambda i: (0, 0))
    brow = pl.BlockSpec((1, D), lambda i: (0, 0))
    return pl.pallas_call(
        _tc_body,
        grid=(nblk,),
        in_specs=[row, row, row, dcol, dcol, full, full, full, brow],
        out_specs=row,
        out_shape=jax.ShapeDtypeStruct((N_NODES, D), jnp.float32),
    )(aggf, aggb, x, degf, degb, w0, w1, wl, bias)


def kernel(x, weight, loop_weight, h_bias, edge_index_fwd, edge_index_bwd):
    n = x.shape[0]
    pad = E_PAD - N_EDGES

    def prep(ei):
        src16 = ei[0].astype(jnp.int32) * NUM_SLABS
        dst = ei[1].astype(jnp.int32)
        src16 = jnp.concatenate([src16, jnp.zeros((pad,), jnp.int32)])
        dst = jnp.concatenate(
            [dst, n + (jnp.arange(pad, dtype=jnp.int32) % 128)])
        return src16.reshape(EDGE_ROWS, 128), dst.reshape(EDGE_ROWS, 128)

    sf, df = prep(edge_index_fwd)
    sb, db = prep(edge_index_bwd)
    src_all = jnp.stack([sf, sb])
    dst_all = jnp.stack([df, db])
    xr = x.reshape(n * NUM_SLABS, SLAB)

    agg_sl, deg = _sc_aggregate(xr, src_all, dst_all)
    agg = agg_sl[:, :, :n, :].transpose(0, 2, 1, 3).reshape(2, n, D)
    degf = deg[0, :n, 0].reshape(n, 1)
    degb = deg[1, :n, 0].reshape(n, 1)
    return _tc_fused(agg[0], agg[1], x, degf, degb,
                     weight[0], weight[1], loop_weight,
                     h_bias.reshape(1, D))


# speedup vs baseline: 3.1987x; 1.0561x over previous
import functools

import jax
import jax.numpy as jnp
from jax import lax
from jax.experimental import pallas as pl
from jax.experimental.pallas import tpu as pltpu
from jax.experimental.pallas import tpu_sc as plsc

N_NODES = 100000
N_EDGES = 1600000
D = 256

NUM_SLABS = 16
SLAB = 16
NT = 16
CHUNK_ROWS = 4
SUPER = 196
TILE_EDGE_ROWS = SUPER * CHUNK_ROWS
EDGE_ROWS = NT * TILE_EDGE_ROWS
E_PAD = EDGE_ROWS * 128
NROW_PAD = 100352
TROWS = NROW_PAD // NT
ZCH = 224
WCH = 112


def _sc_aggregate(xr, src_all, dst_all):
    mesh = plsc.VectorSubcoreMesh(core_axis_name="c", subcore_axis_name="s")

    @functools.partial(
        pl.kernel,
        mesh=mesh,
        compiler_params=pltpu.CompilerParams(use_tc_tiling_on_sc=False),
        out_type=[
            jax.ShapeDtypeStruct((2, NUM_SLABS, NROW_PAD, SLAB), jnp.float32),
            jax.ShapeDtypeStruct((2, NROW_PAD, SLAB), jnp.float32),
        ],
        scratch_types=[
            pltpu.VMEM((2, CHUNK_ROWS, 128), jnp.int32),
            pltpu.VMEM((2, CHUNK_ROWS, 128), jnp.int32),
            pltpu.VMEM((4, CHUNK_ROWS, 128), jnp.int32),
            pltpu.VMEM((2, CHUNK_ROWS, 128, SLAB), jnp.float32),
            pltpu.VMEM((ZCH, SLAB), jnp.float32),
            pltpu.VMEM((128, SLAB), jnp.float32),
            pltpu.VMEM((2, WCH, SLAB), jnp.float32),
            pltpu.VMEM_SHARED((NROW_PAD, SLAB), jnp.float32),
            pltpu.SemaphoreType.DMA,
            pltpu.SemaphoreType.DMA((2,)),
            pltpu.SemaphoreType.DMA((2,)),
            pltpu.SemaphoreType.DMA,
            pltpu.SemaphoreType.DMA,
        ],
    )
    def sc_kernel(xr_hbm, src_hbm, dst_hbm, agg_out, deg_out,
                  src_v, sidx_v, dst_v, rows_v, zero_rows, ones_rows,
                  stage_v, shared_agg, isem, gsem, scsem, rsem, wsem):
        c = lax.axis_index("c")
        tid = lax.axis_index("s")
        row_base = tid * TROWS

        def fill_zr(i, _):
            zero_rows[i, :] = jnp.zeros((SLAB,), jnp.float32)
            return 0
        lax.fori_loop(0, ZCH, fill_zr, 0)

        def fill_on(i, _):
            ones_rows[i, :] = jnp.ones((SLAB,), jnp.float32)
            return 0
        lax.fori_loop(0, 128, fill_on, 0)

        def zero_phase():
            for z in range(TROWS // ZCH):
                pltpu.async_copy(zero_rows,
                                 shared_agg.at[pl.ds(row_base + z * ZCH, ZCH)],
                                 rsem)
            for z in range(TROWS // ZCH):
                pltpu.make_async_copy(
                    zero_rows, shared_agg.at[pl.ds(row_base, ZCH)], rsem).wait()

        def write_phase(out_view):
            nchunk = TROWS // WCH

            def rd(k, kb):
                pltpu.async_copy(
                    shared_agg.at[pl.ds(row_base + k * WCH, WCH)],
                    stage_v.at[kb], rsem)

            def wr_desc(k, kb):
                return pltpu.make_async_copy(
                    stage_v.at[kb], out_view.at[pl.ds(row_base + k * WCH, WCH)],
                    wsem)

            rd(0, 0)

            def wbody(k, _):
                kb = k % 2
                pltpu.make_async_copy(
                    shared_agg.at[pl.ds(row_base, WCH)], stage_v.at[kb],
                    rsem).wait()

                @pl.when(k >= 2)
                def _():
                    wr_desc(k, kb).wait()
                rd((k + 1) % nchunk, 1 - kb)
                wr_desc(k, kb).start()
                return 0
            lax.fori_loop(0, nchunk, wbody, 0)
            wr_desc(nchunk - 2, 0).wait()
            wr_desc(nchunk - 1, 1).wait()
            pltpu.make_async_copy(
                shared_agg.at[pl.ds(row_base, WCH)], stage_v.at[0], rsem).wait()

        def idx_load(rel, i, use_src):
            sc = i % SUPER
            row0 = (tid * SUPER + sc) * CHUNK_ROWS

            @pl.when(use_src)
            def _():
                pltpu.async_copy(src_hbm.at[rel, pl.ds(row0, CHUNK_ROWS)],
                                 src_v.at[i % 2], isem)
            pltpu.async_copy(dst_hbm.at[rel, pl.ds(row0, CHUNK_ROWS)],
                             dst_v.at[i % 4], isem)

        def idx_wait(use_src):
            @pl.when(use_src)
            def _():
                pltpu.make_async_copy(
                    src_hbm.at[0, pl.ds(0, CHUNK_ROWS)], src_v.at[0],
                    isem).wait()
            pltpu.make_async_copy(
                dst_hbm.at[0, pl.ds(0, CHUNK_ROWS)], dst_v.at[0], isem).wait()

        def sc_wait(b):
            for j in range(CHUNK_ROWS):
                pltpu.make_async_copy(
                    rows_v.at[0, j], shared_agg.at[dst_v.at[0, j]],
                    scsem.at[b]).wait()

        def edge_loop(rel, s, gather_on):
            idx_load(rel, 0, gather_on)

            def g_wait(b):
                for j in range(CHUNK_ROWS):
                    pltpu.make_async_copy(
                        xr_hbm.at[sidx_v.at[0, j]], rows_v.at[0, j],
                        gsem.at[b]).wait()

            def scatter_issue(b, d4):
                for j in range(CHUNK_ROWS):
                    pltpu.async_copy(rows_v.at[b, j],
                                     shared_agg.at[dst_v.at[d4, j]],
                                     scsem.at[b], add=True)

            def body(i, _):
                b = i % 2
                d4 = i % 4
                idx_wait(gather_on)
                idx_load(rel, i + 1, gather_on)

                @pl.when(i >= 2)
                def _():
                    sc_wait(b)
                if gather_on:
                    for j in range(CHUNK_ROWS):
                        for l in range(8):
                            sl = pl.ds(l * 16, 16)
                            sidx_v[b, j, sl] = src_v[b, j, sl] + s
                    for j in range(CHUNK_ROWS):
                        pltpu.async_copy(xr_hbm.at[sidx_v.at[b, j]],
                                         rows_v.at[b, j], gsem.at[b])

                    @pl.when(i >= 1)
                    def _():
                        g_wait(1 - b)
                        scatter_issue(1 - b, (i + 3) % 4)
                else:
                    for j in range(CHUNK_ROWS):
                        pltpu.async_copy(ones_rows,
                                         shared_agg.at[dst_v.at[d4, j]],
                                         scsem.at[b], add=True)
                return 0
            lax.fori_loop(0, SUPER, body, 0)
            if gather_on:
                lb = (SUPER - 1) % 2
                g_wait(lb)
                scatter_issue(lb, (SUPER - 1) % 4)
            sc_wait(0)
            sc_wait(1)
            idx_wait(gather_on)

        def pass_body(p, _):
            r = p // 8
            s = c * 8 + (p % 8)
            zero_phase()
            plsc.subcore_barrier()
            edge_loop(r, s, True)
            plsc.subcore_barrier()
            write_phase(agg_out.at[r, s])
            plsc.subcore_barrier()
            return 0

        lax.fori_loop(0, 16, pass_body, 0)

        zero_phase()
        plsc.subcore_barrier()
        edge_loop(c, 0, False)
        plsc.subcore_barrier()
        write_phase(deg_out.at[c])

    return sc_kernel(xr, src_all, dst_all)


def _tc_loop_body(xb, wl, out):
    out[...] = jnp.dot(xb[...], wl[...], preferred_element_type=jnp.float32)


def _tc_loop(x, wl):
    rb = 1000
    row = pl.BlockSpec((rb, D), lambda i: (i, 0))
    full = pl.BlockSpec((D, D), lambda i: (0, 0))
    return pl.pallas_call(
        _tc_loop_body,
        grid=(N_NODES // rb,),
        in_specs=[row, full],
        out_specs=row,
        out_shape=jax.ShapeDtypeStruct((N_NODES, D), jnp.float32),
    )(x, wl)


def _tc_body(aggf, aggb, xw, w0, w1, bias, out):
    acc = jnp.dot(aggf[...], w0[...], preferred_element_type=jnp.float32)
    acc = acc + jnp.dot(aggb[...], w1[...], preferred_element_type=jnp.float32)
    out[...] = jnp.maximum(acc + xw[...] + bias[...], 0.0)


def _tc_fused(aggf, aggb, xw, w0, w1, bias):
    rb = 1000
    nblk = N_NODES // rb
    row = pl.BlockSpec((rb, D), lambda i: (i, 0))
    full = pl.BlockSpec((D, D), lambda i: (0, 0))
    brow = pl.BlockSpec((1, D), lambda i: (0, 0))
    return pl.pallas_call(
        _tc_body,
        grid=(nblk,),
        in_specs=[row, row, row, full, full, brow],
        out_specs=row,
        out_shape=jax.ShapeDtypeStruct((N_NODES, D), jnp.float32),
    )(aggf, aggb, xw, w0, w1, bias)


def kernel(x, weight, loop_weight, h_bias, edge_index_fwd, edge_index_bwd):
    n = x.shape[0]
    pad = E_PAD - N_EDGES

    def prep(ei):
        src16 = ei[0].astype(jnp.int32) * NUM_SLABS
        dst = ei[1].astype(jnp.int32)
        src16 = jnp.concatenate([src16, jnp.zeros((pad,), jnp.int32)])
        dst = jnp.concatenate(
            [dst, n + (jnp.arange(pad, dtype=jnp.int32) % 128)])
        return src16.reshape(EDGE_ROWS, 128), dst.reshape(EDGE_ROWS, 128)

    sf, df = prep(edge_index_fwd)
    sb, db = prep(edge_index_bwd)
    src_all = jnp.stack([sf, sb])
    dst_all = jnp.stack([df, db])
    xr = x.reshape(n * NUM_SLABS, SLAB)

    xw = _tc_loop(x, loop_weight)
    agg_sl, deg = _sc_aggregate(xr, src_all, dst_all)
    norm = 1.0 / jnp.maximum(deg[:, :n, 0], 1.0)
    agg = agg_sl[:, :, :n, :].transpose(0, 2, 1, 3).reshape(2, n, D)
    aggn = agg * norm[:, :, None]
    return _tc_fused(aggn[0], aggn[1], xw,
                     weight[0], weight[1], h_bias.reshape(1, D))


# speedup vs baseline: 5.2596x; 1.6443x over previous
import functools

import jax
import jax.numpy as jnp
from jax import lax
from jax.experimental import pallas as pl
from jax.experimental.pallas import tpu as pltpu
from jax.experimental.pallas import tpu_sc as plsc

N_NODES = 100000
N_EDGES = 1600000
D = 256

NUM_SLABS = 16
SLAB = 16
NT = 16
CHUNK_ROWS = 4
SUPER = 196
TILE_EDGE_ROWS = SUPER * CHUNK_ROWS
EDGE_ROWS = NT * TILE_EDGE_ROWS
E_PAD = EDGE_ROWS * 128
NROW_PAD = 100352
TROWS = NROW_PAD // NT
ZCH = 224
WCH = 112


def _sc_aggregate(xr, src_all, dst_all):
    mesh = plsc.VectorSubcoreMesh(core_axis_name="c", subcore_axis_name="s")

    @functools.partial(
        pl.kernel,
        mesh=mesh,
        compiler_params=pltpu.CompilerParams(use_tc_tiling_on_sc=False),
        out_type=[
            jax.ShapeDtypeStruct((2, NROW_PAD // 8, 2, 8, 128), jnp.float32),
            jax.ShapeDtypeStruct((2, NROW_PAD // 8, 8, 128), jnp.float32),
        ],
        scratch_types=[
            pltpu.VMEM((2, CHUNK_ROWS, 128), jnp.int32),
            pltpu.VMEM((2, CHUNK_ROWS, 128), jnp.int32),
            pltpu.VMEM((4, CHUNK_ROWS, 128), jnp.int32),
            pltpu.VMEM((2, CHUNK_ROWS, 128, SLAB), jnp.float32),
            pltpu.VMEM((ZCH, SLAB), jnp.float32),
            pltpu.VMEM((128, SLAB), jnp.float32),
            pltpu.VMEM((2, WCH // 8, 8, SLAB), jnp.float32),
            pltpu.VMEM_SHARED((NROW_PAD, SLAB), jnp.float32),
            pltpu.SemaphoreType.DMA,
            pltpu.SemaphoreType.DMA((2,)),
            pltpu.SemaphoreType.DMA((2,)),
            pltpu.SemaphoreType.DMA,
            pltpu.SemaphoreType.DMA,
        ],
    )
    def sc_kernel(xr_hbm, src_hbm, dst_hbm, agg_out, deg_out,
                  src_v, sidx_v, dst_v, rows_v, zero_rows, ones_rows,
                  stage_v, shared_agg, isem, gsem, scsem, rsem, wsem):
        c = lax.axis_index("c")
        tid = lax.axis_index("s")
        row_base = tid * TROWS

        def fill_zr(i, _):
            zero_rows[i, :] = jnp.zeros((SLAB,), jnp.float32)
            return 0
        lax.fori_loop(0, ZCH, fill_zr, 0)

        def fill_on(i, _):
            ones_rows[i, :] = jnp.ones((SLAB,), jnp.float32)
            return 0
        lax.fori_loop(0, 128, fill_on, 0)

        def zero_phase():
            for z in range(TROWS // ZCH):
                pltpu.async_copy(zero_rows,
                                 shared_agg.at[pl.ds(row_base + z * ZCH, ZCH)],
                                 rsem)
            for z in range(TROWS // ZCH):
                pltpu.make_async_copy(
                    zero_rows, shared_agg.at[pl.ds(row_base, ZCH)], rsem).wait()

        def write_phase(view_fn):
            nchunk = TROWS // WCH

            def rd(k, kb):
                for g2 in range(WCH // 8):
                    pltpu.async_copy(
                        shared_agg.at[pl.ds(row_base + k * WCH + g2 * 8, 8)],
                        stage_v.at[kb, g2], rsem)

            def rd_wait(kb):
                for g2 in range(WCH // 8):
                    pltpu.make_async_copy(
                        shared_agg.at[pl.ds(row_base, 8)], stage_v.at[kb, g2],
                        rsem).wait()

            def wr_desc(k, kb):
                return pltpu.make_async_copy(
                    stage_v.at[kb], view_fn(k), wsem)

            rd(0, 0)

            def wbody(k, _):
                kb = k % 2
                rd_wait(kb)

                @pl.when(k >= 2)
                def _():
                    wr_desc(k, kb).wait()
                rd((k + 1) % nchunk, 1 - kb)
                wr_desc(k, kb).start()
                return 0
            lax.fori_loop(0, nchunk, wbody, 0)
            wr_desc(nchunk - 2, 0).wait()
            wr_desc(nchunk - 1, 1).wait()
            rd_wait(0)

        def idx_load(rel, i, use_src):
            sc = i % SUPER
            row0 = (tid * SUPER + sc) * CHUNK_ROWS

            @pl.when(use_src)
            def _():
                pltpu.async_copy(src_hbm.at[rel, pl.ds(row0, CHUNK_ROWS)],
                                 src_v.at[i % 2], isem)
            pltpu.async_copy(dst_hbm.at[rel, pl.ds(row0, CHUNK_ROWS)],
                             dst_v.at[i % 4], isem)

        def idx_wait(use_src):
            @pl.when(use_src)
            def _():
                pltpu.make_async_copy(
                    src_hbm.at[0, pl.ds(0, CHUNK_ROWS)], src_v.at[0],
                    isem).wait()
            pltpu.make_async_copy(
                dst_hbm.at[0, pl.ds(0, CHUNK_ROWS)], dst_v.at[0], isem).wait()

        def sc_wait(b):
            for j in range(CHUNK_ROWS):
                pltpu.make_async_copy(
                    rows_v.at[0, j], shared_agg.at[dst_v.at[0, j]],
                    scsem.at[b]).wait()

        def edge_loop(rel, s, gather_on):
            idx_load(rel, 0, gather_on)

            def g_wait(b):
                for j in range(CHUNK_ROWS):
                    pltpu.make_async_copy(
                        xr_hbm.at[sidx_v.at[0, j]], rows_v.at[0, j],
                        gsem.at[b]).wait()

            def scatter_issue(b, d4):
                for j in range(CHUNK_ROWS):
                    pltpu.async_copy(rows_v.at[b, j],
                                     shared_agg.at[dst_v.at[d4, j]],
                                     scsem.at[b], add=True)

            def body(i, _):
                b = i % 2
                d4 = i % 4
                idx_wait(gather_on)
                idx_load(rel, i + 1, gather_on)

                @pl.when(i >= 2)
                def _():
                    sc_wait(b)
                if gather_on:
                    for j in range(CHUNK_ROWS):
                        for l in range(8):
                            sl = pl.ds(l * 16, 16)
                            sidx_v[b, j, sl] = src_v[b, j, sl] + s
                    for j in range(CHUNK_ROWS):
                        pltpu.async_copy(xr_hbm.at[sidx_v.at[b, j]],
                                         rows_v.at[b, j], gsem.at[b])

                    @pl.when(i >= 1)
                    def _():
                        g_wait(1 - b)
                        scatter_issue(1 - b, (i + 3) % 4)
                else:
                    for j in range(CHUNK_ROWS):
                        pltpu.async_copy(ones_rows,
                                         shared_agg.at[dst_v.at[d4, j]],
                                         scsem.at[b], add=True)
                return 0
            lax.fori_loop(0, SUPER, body, 0)
            if gather_on:
                lb = (SUPER - 1) % 2
                g_wait(lb)
                scatter_issue(lb, (SUPER - 1) % 4)
            sc_wait(0)
            sc_wait(1)
            idx_wait(gather_on)

        def pass_body(p, _):
            r = p // 8
            si = p % 8
            s = c * 8 + si
            zero_phase()
            plsc.subcore_barrier()
            edge_loop(r, s, True)
            plsc.subcore_barrier()

            def agg_view(k):
                off8 = (row_base + k * WCH) // 8
                return agg_out.at[r, pl.ds(off8, WCH // 8), c, :,
                                  pl.ds(si * SLAB, SLAB)]
            write_phase(agg_view)
            plsc.subcore_barrier()
            return 0

        lax.fori_loop(0, 16, pass_body, 0)

        zero_phase()
        plsc.subcore_barrier()
        edge_loop(c, 0, False)
        plsc.subcore_barrier()

        def deg_view(k):
            off8 = (row_base + k * WCH) // 8
            return deg_out.at[c, pl.ds(off8, WCH // 8), :, pl.ds(0, SLAB)]
        write_phase(deg_view)

    return sc_kernel(xr, src_all, dst_all)


def _tc_loop_body(xb, wl, out):
    out[...] = jnp.dot(xb[...], wl[...], preferred_element_type=jnp.float32)


def _tc_loop(x, wl):
    rb = 1000
    row = pl.BlockSpec((rb, D), lambda i: (i, 0))
    full = pl.BlockSpec((D, D), lambda i: (0, 0))
    return pl.pallas_call(
        _tc_loop_body,
        grid=(N_NODES // rb,),
        in_specs=[row, full],
        out_specs=row,
        out_shape=jax.ShapeDtypeStruct((N_NODES, D), jnp.float32),
    )(x, wl)


RB = 1000


def _tc_body(aggf, aggb, degf, degb, xw, w0, w1, bias, out):
    def half(ref, t):
        return ref[...][:, t, :, :].reshape(RB, 128)

    def norm(ref):
        return 1.0 / jnp.maximum(ref[...][:, :, 0:1].reshape(RB, 1), 1.0)

    nf = norm(degf)
    nb = norm(degb)
    w0m = w0[...]
    w1m = w1[...]
    acc = jnp.dot(half(aggf, 0) * nf, w0m[0:128, :],
                  preferred_element_type=jnp.float32)
    acc = acc + jnp.dot(half(aggf, 1) * nf, w0m[128:256, :],
                        preferred_element_type=jnp.float32)
    acc = acc + jnp.dot(half(aggb, 0) * nb, w1m[0:128, :],
                        preferred_element_type=jnp.float32)
    acc = acc + jnp.dot(half(aggb, 1) * nb, w1m[128:256, :],
                        preferred_element_type=jnp.float32)
    out[...] = jnp.maximum(acc + xw[...] + bias[...], 0.0)


def _tc_fused(aggf5, aggb5, degf5, degb5, xw, w0, w1, bias):
    g = RB // 8
    agg_spec = pl.BlockSpec((g, 2, 8, 128), lambda i: (i, 0, 0, 0))
    deg_spec = pl.BlockSpec((g, 8, 128), lambda i: (i, 0, 0))
    row = pl.BlockSpec((RB, D), lambda i: (i, 0))
    full = pl.BlockSpec((D, D), lambda i: (0, 0))
    brow = pl.BlockSpec((1, D), lambda i: (0, 0))
    return pl.pallas_call(
        _tc_body,
        grid=(N_NODES // RB,),
        in_specs=[agg_spec, agg_spec, deg_spec, deg_spec, row, full, full,
                  brow],
        out_specs=row,
        out_shape=jax.ShapeDtypeStruct((N_NODES, D), jnp.float32),
    )(aggf5, aggb5, degf5, degb5, xw, w0, w1, bias)


def kernel(x, weight, loop_weight, h_bias, edge_index_fwd, edge_index_bwd):
    n = x.shape[0]
    pad = E_PAD - N_EDGES

    def prep(ei):
        src16 = ei[0].astype(jnp.int32) * NUM_SLABS
        dst = ei[1].astype(jnp.int32)
        src16 = jnp.concatenate([src16, jnp.zeros((pad,), jnp.int32)])
        dst = jnp.concatenate(
            [dst, n + (jnp.arange(pad, dtype=jnp.int32) % 128)])
        return src16.reshape(EDGE_ROWS, 128), dst.reshape(EDGE_ROWS, 128)

    sf, df = prep(edge_index_fwd)
    sb, db = prep(edge_index_bwd)
    src_all = jnp.stack([sf, sb])
    dst_all = jnp.stack([df, db])
    xr = x.reshape(n * NUM_SLABS, SLAB)

    xw = _tc_loop(x, loop_weight)
    agg5, deg5 = _sc_aggregate(xr, src_all, dst_all)
    return _tc_fused(agg5[0], agg5[1], deg5[0], deg5[1], xw,
                     weight[0], weight[1], h_bias.reshape(1, D))


# speedup vs baseline: 5.2864x; 1.0051x over previous
import functools

import jax
import jax.numpy as jnp
from jax import lax
from jax.experimental import pallas as pl
from jax.experimental.pallas import tpu as pltpu
from jax.experimental.pallas import tpu_sc as plsc

N_NODES = 100000
N_EDGES = 1600000
D = 256

NUM_SLABS = 16
SLAB = 16
NT = 16
CHUNK_ROWS = 4
SUPER = 196
TILE_EDGE_ROWS = SUPER * CHUNK_ROWS
EDGE_ROWS = NT * TILE_EDGE_ROWS
E_PAD = EDGE_ROWS * 128
NROW_PAD = 100352
TROWS = NROW_PAD // NT
ZCH = 224
WCH = 112


def _sc_aggregate(xr, src_all, dst_all):
    mesh = plsc.VectorSubcoreMesh(core_axis_name="c", subcore_axis_name="s")

    @functools.partial(
        pl.kernel,
        mesh=mesh,
        compiler_params=pltpu.CompilerParams(use_tc_tiling_on_sc=False),
        out_type=[
            jax.ShapeDtypeStruct((2, NROW_PAD // 8, 2, 8, 128), jnp.float32),
            jax.ShapeDtypeStruct((2, NROW_PAD // 8, 8, 128), jnp.float32),
        ],
        scratch_types=[
            pltpu.VMEM((2, CHUNK_ROWS * 128), jnp.int32),
            pltpu.VMEM((2, CHUNK_ROWS * 128), jnp.int32),
            pltpu.VMEM((4, CHUNK_ROWS, 128), jnp.int32),
            pltpu.VMEM((2, CHUNK_ROWS * 128, SLAB), jnp.float32),
            pltpu.VMEM((ZCH, SLAB), jnp.float32),
            pltpu.VMEM((128, SLAB), jnp.float32),
            pltpu.VMEM((2, WCH // 8, 8, SLAB), jnp.float32),
            pltpu.VMEM_SHARED((NROW_PAD, SLAB), jnp.float32),
            pltpu.SemaphoreType.DMA,
            pltpu.SemaphoreType.DMA((2,)),
            pltpu.SemaphoreType.DMA((2,)),
            pltpu.SemaphoreType.DMA,
            pltpu.SemaphoreType.DMA,
        ],
    )
    def sc_kernel(xr_hbm, src_hbm, dst_hbm, agg_out, deg_out,
                  src_v, sidx_v, dst_v, rows_v, zero_rows, ones_rows,
                  stage_v, shared_agg, isem, gsem, scsem, rsem, wsem):
        c = lax.axis_index("c")
        tid = lax.axis_index("s")
        row_base = tid * TROWS

        def fill_zr(i, _):
            zero_rows[i, :] = jnp.zeros((SLAB,), jnp.float32)
            return 0
        lax.fori_loop(0, ZCH, fill_zr, 0)

        def fill_on(i, _):
            ones_rows[i, :] = jnp.ones((SLAB,), jnp.float32)
            return 0
        lax.fori_loop(0, 128, fill_on, 0)

        def zero_phase():
            for z in range(TROWS // ZCH):
                pltpu.async_copy(zero_rows,
                                 shared_agg.at[pl.ds(row_base + z * ZCH, ZCH)],
                                 rsem)
            for z in range(TROWS // ZCH):
                pltpu.make_async_copy(
                    zero_rows, shared_agg.at[pl.ds(row_base, ZCH)], rsem).wait()

        def write_phase(view_fn):
            nchunk = TROWS // WCH

            def rd(k, kb):
                for g2 in range(WCH // 8):
                    pltpu.async_copy(
                        shared_agg.at[pl.ds(row_base + k * WCH + g2 * 8, 8)],
                        stage_v.at[kb, g2], rsem)

            def rd_wait(kb):
                for g2 in range(WCH // 8):
                    pltpu.make_async_copy(
                        shared_agg.at[pl.ds(row_base, 8)], stage_v.at[kb, g2],
                        rsem).wait()

            def wr_desc(k, kb):
                return pltpu.make_async_copy(
                    stage_v.at[kb], view_fn(k), wsem)

            rd(0, 0)

            def wbody(k, _):
                kb = k % 2
                rd_wait(kb)

                @pl.when(k >= 2)
                def _():
                    wr_desc(k, kb).wait()
                rd((k + 1) % nchunk, 1 - kb)
                wr_desc(k, kb).start()
                return 0
            lax.fori_loop(0, nchunk, wbody, 0)
            wr_desc(nchunk - 2, 0).wait()
            wr_desc(nchunk - 1, 1).wait()
            rd_wait(0)

        def idx_load(rel, i, use_src):
            sc = i % SUPER
            row0 = (tid * SUPER + sc) * CHUNK_ROWS

            @pl.when(use_src)
            def _():
                pltpu.async_copy(
                    src_hbm.at[rel, pl.ds(row0 * 128, CHUNK_ROWS * 128)],
                    src_v.at[i % 2], isem)
            pltpu.async_copy(dst_hbm.at[rel, pl.ds(row0, CHUNK_ROWS)],
                             dst_v.at[i % 4], isem)

        def idx_wait(use_src):
            @pl.when(use_src)
            def _():
                pltpu.make_async_copy(
                    src_hbm.at[0, pl.ds(0, CHUNK_ROWS * 128)], src_v.at[0],
                    isem).wait()
            pltpu.make_async_copy(
                dst_hbm.at[0, pl.ds(0, CHUNK_ROWS)], dst_v.at[0], isem).wait()

        def sc_wait(b):
            for j in range(CHUNK_ROWS):
                pltpu.make_async_copy(
                    rows_v.at[0, pl.ds(0, 128)], shared_agg.at[dst_v.at[0, j]],
                    scsem.at[b]).wait()

        def edge_loop(rel, s, gather_on):
            idx_load(rel, 0, gather_on)

            def g_wait(b):
                pltpu.make_async_copy(
                    xr_hbm.at[sidx_v.at[0]], rows_v.at[0],
                    gsem.at[b]).wait()

            def scatter_issue(b, d4):
                for j in range(CHUNK_ROWS):
                    pltpu.async_copy(rows_v.at[b, pl.ds(j * 128, 128)],
                                     shared_agg.at[dst_v.at[d4, j]],
                                     scsem.at[b], add=True)

            def body(i, _):
                b = i % 2
                d4 = i % 4
                idx_wait(gather_on)
                idx_load(rel, i + 1, gather_on)

                @pl.when(i >= 2)
                def _():
                    sc_wait(b)
                if gather_on:
                    for l in range(CHUNK_ROWS * 8):
                        sl = pl.ds(l * 16, 16)
                        sidx_v[b, sl] = src_v[b, sl] + s
                    pltpu.async_copy(xr_hbm.at[sidx_v.at[b]],
                                     rows_v.at[b], gsem.at[b])

                    @pl.when(i >= 1)
                    def _():
                        g_wait(1 - b)
                        scatter_issue(1 - b, (i + 3) % 4)
                else:
                    for j in range(CHUNK_ROWS):
                        pltpu.async_copy(ones_rows,
                                         shared_agg.at[dst_v.at[d4, j]],
                                         scsem.at[b], add=True)
                return 0
            lax.fori_loop(0, SUPER, body, 0)
            if gather_on:
                lb = (SUPER - 1) % 2
                g_wait(lb)
                scatter_issue(lb, (SUPER - 1) % 4)
            sc_wait(0)
            sc_wait(1)
            idx_wait(gather_on)

        def pass_body(p, _):
            r = p // 8
            si = p % 8
            s = c * 8 + si
            zero_phase()
            plsc.subcore_barrier()
            edge_loop(r, s, True)
            plsc.subcore_barrier()

            def agg_view(k):
                off8 = (row_base + k * WCH) // 8
                return agg_out.at[r, pl.ds(off8, WCH // 8), c, :,
                                  pl.ds(si * SLAB, SLAB)]
            write_phase(agg_view)
            plsc.subcore_barrier()
            return 0

        lax.fori_loop(0, 16, pass_body, 0)

        zero_phase()
        plsc.subcore_barrier()
        edge_loop(c, 0, False)
        plsc.subcore_barrier()

        def deg_view(k):
            off8 = (row_base + k * WCH) // 8
            return deg_out.at[c, pl.ds(off8, WCH // 8), :, pl.ds(0, SLAB)]
        write_phase(deg_view)

    return sc_kernel(xr, src_all, dst_all)


def _tc_loop_body(xb, wl, out):
    out[...] = jnp.dot(xb[...], wl[...], preferred_element_type=jnp.float32)


def _tc_loop(x, wl):
    rb = 1000
    row = pl.BlockSpec((rb, D), lambda i: (i, 0))
    full = pl.BlockSpec((D, D), lambda i: (0, 0))
    return pl.pallas_call(
        _tc_loop_body,
        grid=(N_NODES // rb,),
        in_specs=[row, full],
        out_specs=row,
        out_shape=jax.ShapeDtypeStruct((N_NODES, D), jnp.float32),
    )(x, wl)


RB = 1000


def _tc_body(aggf, aggb, degf, degb, xw, w0, w1, bias, out):
    def half(ref, t):
        return ref[...][:, t, :, :].reshape(RB, 128)

    def norm(ref):
        return 1.0 / jnp.maximum(ref[...][:, :, 0:1].reshape(RB, 1), 1.0)

    nf = norm(degf)
    nb = norm(degb)
    w0m = w0[...]
    w1m = w1[...]
    acc = jnp.dot(half(aggf, 0) * nf, w0m[0:128, :],
                  preferred_element_type=jnp.float32)
    acc = acc + jnp.dot(half(aggf, 1) * nf, w0m[128:256, :],
                        preferred_element_type=jnp.float32)
    acc = acc + jnp.dot(half(aggb, 0) * nb, w1m[0:128, :],
                        preferred_element_type=jnp.float32)
    acc = acc + jnp.dot(half(aggb, 1) * nb, w1m[128:256, :],
                        preferred_element_type=jnp.float32)
    out[...] = jnp.maximum(acc + xw[...] + bias[...], 0.0)


def _tc_fused(aggf5, aggb5, degf5, degb5, xw, w0, w1, bias):
    g = RB // 8
    agg_spec = pl.BlockSpec((g, 2, 8, 128), lambda i: (i, 0, 0, 0))
    deg_spec = pl.BlockSpec((g, 8, 128), lambda i: (i, 0, 0))
    row = pl.BlockSpec((RB, D), lambda i: (i, 0))
    full = pl.BlockSpec((D, D), lambda i: (0, 0))
    brow = pl.BlockSpec((1, D), lambda i: (0, 0))
    return pl.pallas_call(
        _tc_body,
        grid=(N_NODES // RB,),
        in_specs=[agg_spec, agg_spec, deg_spec, deg_spec, row, full, full,
                  brow],
        out_specs=row,
        out_shape=jax.ShapeDtypeStruct((N_NODES, D), jnp.float32),
    )(aggf5, aggb5, degf5, degb5, xw, w0, w1, bias)


def kernel(x, weight, loop_weight, h_bias, edge_index_fwd, edge_index_bwd):
    n = x.shape[0]
    pad = E_PAD - N_EDGES

    def prep(ei):
        src16 = ei[0].astype(jnp.int32) * NUM_SLABS
        dst = ei[1].astype(jnp.int32)
        src16 = jnp.concatenate([src16, jnp.zeros((pad,), jnp.int32)])
        dst = jnp.concatenate(
            [dst, n + (jnp.arange(pad, dtype=jnp.int32) % 128)])
        return src16, dst.reshape(EDGE_ROWS, 128)

    sf, df = prep(edge_index_fwd)
    sb, db = prep(edge_index_bwd)
    src_all = jnp.stack([sf, sb])
    dst_all = jnp.stack([df, db])
    xr = x.reshape(n * NUM_SLABS, SLAB)

    xw = _tc_loop(x, loop_weight)
    agg5, deg5 = _sc_aggregate(xr, src_all, dst_all)
    return _tc_fused(agg5[0], agg5[1], deg5[0], deg5[1], xw,
                     weight[0], weight[1], h_bias.reshape(1, D))
